# baseline (speedup 1.0000x reference)
import jax
import jax.numpy as jnp
from jax import lax
from jax.experimental import pallas as pl
from jax.experimental.pallas import tpu as pltpu
from jax.experimental.pallas import tpu_sc as plsc

NUM_COLLISIONS = 1000
EMBED = 64
WORDS = EMBED // 2
L = 16
NC, NS = 2, 16
NW = NC * NS
BATCH, FIELDS = 16384, 26
TOTAL = BATCH * FIELDS
PER_W = TOTAL // NW
CHUNK = 128
N_CHUNKS = PER_W // CHUNK
RING = 4
N_GROUPS = N_CHUNKS // RING
SLICES = CHUNK // L


def _qr_body(idx_hbm, wqp_hbm, wrp_hbm, out_hbm,
             idx_all, wq_loc, wr_loc, prod, sem_o):
    wid = lax.axis_index("s") * NC + lax.axis_index("c")
    obase = wid * PER_W

    pltpu.sync_copy(wqp_hbm, wq_loc)
    pltpu.sync_copy(wrp_hbm, wr_loc)
    pltpu.sync_copy(idx_hbm.at[wid], idx_all)

    def pack_body(i, carry):
        row = lax.div(i, SLICES)
        col = lax.rem(i, SLICES) * L
        v = idx_all[row, pl.ds(col, L)]
        vf = (v.astype(jnp.float32) + 0.5) * jnp.float32(1.0 / NUM_COLLISIONS)
        q = vf.astype(jnp.int32)
        r = v - q * NUM_COLLISIONS
        idx_all[row, pl.ds(col, L)] = q * 65536 + r
        return carry
    lax.fori_loop(0, N_CHUNKS * SLICES, pack_body, 0)

    def group_body(g, carry):
        for b in range(RING):
            cg = g * RING + b
            @pl.when(g > 0)
            def _():
                pltpu.make_async_copy(
                    prod.at[b], out_hbm.at[pl.ds(obase, CHUNK)], sem_o.at[b]
                ).wait()

            @plsc.parallel_loop(0, SLICES, 1, unroll=2)
            def _(j):
                c16 = idx_all[cg, pl.ds(j * L, L)]
                qs16 = lax.shift_right_logical(c16, 16)
                rs16 = lax.bitwise_and(c16, 65535)
                for l in range(L):
                    qs, rs = qs16[l], rs16[l]
                    i = j * L + l
                    for s in range(2):
                        qw = plsc.bitcast(wq_loc[qs, pl.ds(s * L, L)],
                                          jnp.bfloat16)
                        rw = plsc.bitcast(wr_loc[rs, pl.ds(s * L, L)],
                                          jnp.bfloat16)
                        pa, pb = plsc.unpack(
                            qw * rw, format=plsc.PackFormat.INTERLEAVED)
                        prod[b, i, pl.ds(s * 2 * L, L)] = pa
                        prod[b, i, pl.ds((s * 2 + 1) * L, L)] = pb

            pltpu.async_copy(
                prod.at[b], out_hbm.at[pl.ds(obase + cg * CHUNK, CHUNK)],
                sem_o.at[b])
        return carry

    lax.fori_loop(0, N_GROUPS, group_body, 0)

    for b in range(RING):
        pltpu.make_async_copy(
            prod.at[b], out_hbm.at[pl.ds(obase, CHUNK)], sem_o.at[b]).wait()


@jax.jit
def _qr_embed(idx_3d, wq_packed, wr_packed):
    mesh = plsc.VectorSubcoreMesh(core_axis_name="c", subcore_axis_name="s")
    return pl.kernel(
        _qr_body,
        out_type=jax.ShapeDtypeStruct((TOTAL, EMBED), jnp.float32),
        mesh=mesh,
        scratch_types=[
            pltpu.VMEM((N_CHUNKS, CHUNK), jnp.int32),
            pltpu.VMEM((NUM_COLLISIONS, WORDS), jnp.int32),
            pltpu.VMEM((NUM_COLLISIONS, WORDS), jnp.int32),
            pltpu.VMEM((RING, CHUNK, EMBED), jnp.float32),
            pltpu.SemaphoreType.DMA((RING,)),
        ],
        compiler_params=pltpu.CompilerParams(
            use_tc_tiling_on_sc=False, needs_layout_passes=False),
    )(idx_3d, wq_packed, wr_packed)


def _pack_table(w):
    u = lax.bitcast_convert_type(w.astype(jnp.bfloat16), jnp.uint16)
    u = u.astype(jnp.uint32).reshape(w.shape[0], 2, 2, L)
    packed = u[:, :, 0, :] | (u[:, :, 1, :] << 16)
    return lax.bitcast_convert_type(packed, jnp.int32).reshape(w.shape[0], WORDS)


def kernel(indices, weight_q, weight_r):
    idx_3d = indices.reshape(NW, N_CHUNKS, CHUNK)
    out = _qr_embed(idx_3d, _pack_table(weight_q), _pack_table(weight_r))
    return out.reshape(BATCH, FIELDS, EMBED)


# speedup vs baseline: 1.2642x; 1.2642x over previous
import jax
import jax.numpy as jnp
from jax import lax
from jax.experimental import pallas as pl
from jax.experimental.pallas import tpu as pltpu
from jax.experimental.pallas import tpu_sc as plsc

NUM_COLLISIONS = 1000
EMBED = 64
WORDS = EMBED // 2
L = 16
NC, NS = 2, 16
NW = NC * NS
BATCH, FIELDS = 16384, 26
TOTAL = BATCH * FIELDS
PER_W = TOTAL // NW
CHUNK = 128
N_CHUNKS = PER_W // CHUNK
RING = 4
N_GROUPS = N_CHUNKS // RING
SLICES = CHUNK // L


def _qr_body(idx_hbm, wqp_hbm, wrp_hbm, out_hbm,
             idx_all, qv, rv, rq, rr, prod, wq_sh, wr_sh, sem_g, sem_o):
    wid = lax.axis_index("s") * NC + lax.axis_index("c")
    obase = wid * PER_W

    @pl.when(lax.axis_index("s") == 0)
    def _():
        pltpu.sync_copy(wqp_hbm, wq_sh)
        pltpu.sync_copy(wrp_hbm, wr_sh)

    pltpu.sync_copy(idx_hbm.at[wid], idx_all)
    plsc.subcore_barrier()

    def pack_body(i, carry):
        row = lax.div(i, SLICES)
        col = lax.rem(i, SLICES) * L
        v = idx_all[row, pl.ds(col, L)]
        vf = (v.astype(jnp.float32) + 0.5) * jnp.float32(1.0 / NUM_COLLISIONS)
        q = vf.astype(jnp.int32)
        r = v - q * NUM_COLLISIONS
        idx_all[row, pl.ds(col, L)] = q * 65536 + r
        return carry
    lax.fori_loop(0, N_CHUNKS * SLICES, pack_body, 0)

    def unpack_and_fire(cg, b):
        for s in range(SLICES):
            c = idx_all[cg, pl.ds(s * L, L)]
            qv[b, pl.ds(s * L, L)] = lax.shift_right_logical(c, 16)
            rv[b, pl.ds(s * L, L)] = lax.bitwise_and(c, 65535)
        pltpu.async_copy(wq_sh.at[qv.at[b]], rq.at[b], sem_g.at[b])
        pltpu.async_copy(wr_sh.at[rv.at[b]], rr.at[b], sem_g.at[b])

    for b in range(RING):
        unpack_and_fire(b, b)

    def group_body(g, carry):
        for b in range(RING):
            cg = g * RING + b
            @pl.when(g > 0)
            def _():
                pltpu.make_async_copy(
                    prod.at[b], out_hbm.at[pl.ds(obase, CHUNK)], sem_o.at[b]
                ).wait()
            pltpu.make_async_copy(wq_sh.at[qv.at[b]], rq.at[b],
                                  sem_g.at[b]).wait()
            pltpu.make_async_copy(wr_sh.at[rv.at[b]], rr.at[b],
                                  sem_g.at[b]).wait()

            @plsc.parallel_loop(0, CHUNK, 1, unroll=4)
            def _(i):
                for s in range(2):
                    cq = plsc.bitcast(rq[b, i, pl.ds(s * L, L)], jnp.bfloat16)
                    cr = plsc.bitcast(rr[b, i, pl.ds(s * L, L)], jnp.bfloat16)
                    qa, qb = plsc.unpack(cq, format=plsc.PackFormat.INTERLEAVED)
                    ra, rb = plsc.unpack(cr, format=plsc.PackFormat.INTERLEAVED)
                    prod[b, i, pl.ds(s * 2 * L, L)] = qa * ra
                    prod[b, i, pl.ds((s * 2 + 1) * L, L)] = qb * rb

            pltpu.async_copy(
                prod.at[b], out_hbm.at[pl.ds(obase + cg * CHUNK, CHUNK)],
                sem_o.at[b])

            @pl.when(g < N_GROUPS - 1)
            def _():
                unpack_and_fire(cg + RING, b)
        return carry

    lax.fori_loop(0, N_GROUPS, group_body, 0)

    for b in range(RING):
        pltpu.make_async_copy(
            prod.at[b], out_hbm.at[pl.ds(obase, CHUNK)], sem_o.at[b]).wait()


@jax.jit
def _qr_embed(idx_3d, wq_packed, wr_packed):
    mesh = plsc.VectorSubcoreMesh(core_axis_name="c", subcore_axis_name="s")
    return pl.kernel(
        _qr_body,
        out_type=jax.ShapeDtypeStruct((TOTAL, EMBED), jnp.float32),
        mesh=mesh,
        scratch_types=[
            pltpu.VMEM((N_CHUNKS, CHUNK), jnp.int32),
            pltpu.VMEM((RING, CHUNK), jnp.int32),
            pltpu.VMEM((RING, CHUNK), jnp.int32),
            pltpu.VMEM((RING, CHUNK, WORDS), jnp.int32),
            pltpu.VMEM((RING, CHUNK, WORDS), jnp.int32),
            pltpu.VMEM((RING, CHUNK, EMBED), jnp.float32),
            pltpu.VMEM_SHARED((NUM_COLLISIONS, WORDS), jnp.int32),
            pltpu.VMEM_SHARED((NUM_COLLISIONS, WORDS), jnp.int32),
            pltpu.SemaphoreType.DMA((RING,)),
            pltpu.SemaphoreType.DMA((RING,)),
        ],
        compiler_params=pltpu.CompilerParams(
            use_tc_tiling_on_sc=False, needs_layout_passes=False),
    )(idx_3d, wq_packed, wr_packed)


def _pack_table(w):
    u = lax.bitcast_convert_type(w.astype(jnp.bfloat16), jnp.uint16)
    u = u.astype(jnp.uint32).reshape(w.shape[0], 2, 2, L)
    packed = u[:, :, 0, :] | (u[:, :, 1, :] << 16)
    return lax.bitcast_convert_type(packed, jnp.int32).reshape(w.shape[0], WORDS)


def kernel(indices, weight_q, weight_r):
    idx_3d = indices.reshape(NW, N_CHUNKS, CHUNK)
    out = _qr_embed(idx_3d, _pack_table(weight_q), _pack_table(weight_r))
    return out.reshape(BATCH, FIELDS, EMBED)
